# baseline (speedup 1.0000x reference)
import functools

import jax
import jax.numpy as jnp
from jax import lax
from jax.experimental import pallas as pl
from jax.experimental.pallas import tpu as pltpu
from jax.experimental.pallas import tpu_sc as plsc

NC = 2
NS = 16
NW = NC * NS

D = 64
CH = 128
NBUF = 4
L = 16
GP = 129


@functools.partial(jax.jit, static_argnames=("v",))
def _relayout(table_t, tail_t, *, v):
    n_cols = v // CH
    n_tail = v - n_cols * CH
    per_w = -(-n_cols // NW)
    n_grp = -(-per_w // NBUF)
    mesh = plsc.VectorSubcoreMesh(core_axis_name="c", subcore_axis_name="s")

    @functools.partial(
        pl.kernel,
        out_type=jax.ShapeDtypeStruct((v // 2, 2 * D), jnp.float32),
        mesh=mesh,
        scratch_types=[
            [pltpu.VMEM((D, GP), jnp.float32) for _ in range(NBUF)],
            [pltpu.VMEM((D, CH), jnp.float32) for _ in range(NBUF)],
            [pltpu.SemaphoreType.DMA for _ in range(NBUF)],
            [pltpu.SemaphoreType.DMA for _ in range(NBUF)],
        ],
        compiler_params=pltpu.CompilerParams(needs_layout_passes=False),
    )
    def k(tab_hbm, tail_hbm, out_hbm, gbufs, tbufs, gsems, wsems):
        wid = lax.axis_index("s") * NC + lax.axis_index("c")
        lo = wid * per_w
        n_my = jnp.clip(n_cols - lo, 0, per_w)
        lane = lax.iota(jnp.int32, L)

        def read_col(c, b):
            return pltpu.make_async_copy(
                tab_hbm.at[:, pl.ds(c * CH, CH)],
                gbufs[b].at[:, pl.ds(0, CH)], gsems[b])

        def write_col(c, b):
            return pltpu.make_async_copy(
                tbufs[b], out_hbm.at[pl.ds(c * D, D)], wsems[b])

        def transpose_col(b, np_rows):
            gb, tb = gbufs[b], tbufs[b]

            @plsc.parallel_loop(0, np_rows, step=1)
            def _(p):
                for u in range(2):
                    col = jnp.full((L,), 2 * p + u, dtype=jnp.int32)
                    for dd in range(D // L):
                        val = plsc.load_gather(gb, [lane + (L * dd), col])
                        tb[p, pl.ds(D * u + L * dd, L)] = val

        for b in range(NBUF - 1):
            @pl.when(b < n_my)
            def _():
                read_col(lo + b, b).start()

        def body(i, carry):
            so = i * NBUF
            for b in range(NBUF):
                g = so + b

                @pl.when(g < n_my)
                def _():
                    @pl.when(g >= NBUF)
                    def _():
                        write_col(lo + g - NBUF, b).wait()

                    read_col(lo + g, b).wait()
                    bn = (b + NBUF - 1) % NBUF

                    @pl.when(g + NBUF - 1 < n_my)
                    def _():
                        read_col(lo + g + NBUF - 1, bn).start()

                    transpose_col(b, D)
                    write_col(lo + g, b).start()
            return carry

        lax.fori_loop(0, n_grp, body, 0)

        for b in range(NBUF):
            g_b = n_my - 1 - ((n_my - 1 - b) % NBUF)

            @pl.when(g_b >= 0)
            def _():
                write_col(lo + g_b, b).wait()

        if n_tail:
            @pl.when(wid == 0)
            def _():
                pltpu.sync_copy(tail_hbm, gbufs[0].at[:, pl.ds(0, CH)])
                transpose_col(0, n_tail // 2)
                pltpu.sync_copy(
                    tbufs[0].at[pl.ds(0, n_tail // 2)],
                    out_hbm.at[pl.ds(n_cols * D, n_tail // 2)])

    return k(table_t, tail_t)


@functools.partial(jax.jit, static_argnames=("n_h", "n_b"))
def _gather(idx3, table_flat, *, n_h, n_b):
    n_rows = n_h * n_b
    per_w = n_rows // NW
    n_ch = per_w // CH
    mesh = plsc.VectorSubcoreMesh(core_axis_name="c", subcore_axis_name="s")

    @functools.partial(
        pl.kernel,
        out_type=jax.ShapeDtypeStruct((n_h, D, n_b), jnp.float32),
        mesh=mesh,
        scratch_types=[
            pltpu.VMEM((n_ch, CH), jnp.int32),
            [pltpu.VMEM((CH, D), jnp.float32) for _ in range(NBUF)],
            [pltpu.VMEM((D, CH), jnp.float32) for _ in range(NBUF)],
            [pltpu.SemaphoreType.DMA for _ in range(NBUF)],
            [pltpu.SemaphoreType.DMA for _ in range(NBUF)],
        ],
        compiler_params=pltpu.CompilerParams(
            use_tc_tiling_on_sc=False, needs_layout_passes=False),
    )
    def k(table_hbm, idx_hbm, out_hbm, idx_v, gbufs, tbufs, gsems, wsems):
        wid = lax.axis_index("s") * NC + lax.axis_index("c")
        w_base = wid * per_w

        pltpu.sync_copy(idx_hbm.at[wid], idx_v)

        def gather_chunk(g, b):
            return pltpu.make_async_copy(
                table_hbm.at[idx_v.at[g]], gbufs[b], gsems[b])

        def write_chunk(g, b):
            row = w_base + g * CH
            return pltpu.make_async_copy(
                tbufs[b],
                out_hbm.at[row // n_b, :, pl.ds(row % n_b, CH)],
                wsems[b])

        def transpose_chunk(b):
            gb, tb = gbufs[b], tbufs[b]
            lane = lax.iota(jnp.int32, L)

            @plsc.parallel_loop(0, (D // L) * (CH // L), step=1)
            def _(i):
                d0 = (i & ((D // L) - 1)) * L
                rows = lane + (i >> 2) * L
                for s in range(L):
                    colv = ((lane + s) & (L - 1)) + d0
                    val = plsc.load_gather(gb, [rows, colv])
                    plsc.store_scatter(tb, [colv, rows], val)

        for b in range(NBUF - 1):
            gather_chunk(b, b).start()

        def outer(i, carry):
            so = i * NBUF
            for b in range(NBUF):
                g = so + b
                @pl.when(g >= NBUF)
                def _():
                    write_chunk(g - NBUF, b).wait()

                gather_chunk(g, b).wait()
                bn = (b + NBUF - 1) % NBUF

                @pl.when(g + NBUF - 1 < n_ch)
                def _():
                    gather_chunk(g + NBUF - 1, bn).start()

                transpose_chunk(b)
                write_chunk(g, b).start()
            return carry

        lax.fori_loop(0, n_ch // NBUF, outer, 0)

        for j in range(NBUF):
            g = n_ch - NBUF + j
            write_chunk(g, g % NBUF).wait()

    return k(table_flat, idx3)


def kernel(imputs, table):
    b, h = imputs.shape
    n_rows = b * h
    per_w = n_rows // NW
    v = table.shape[0]
    idx3 = imputs.T.reshape(NW, per_w // CH, CH).astype(jnp.int32)
    n_tail = v - (v // CH) * CH
    tail_t = jnp.pad(table[v - n_tail:].T, ((0, 0), (0, CH - n_tail)))
    table_flat = _relayout(table.T, tail_t, v=v).reshape(v, D)
    out = _gather(idx3, table_flat, n_h=h, n_b=b)
    return out.transpose(2, 0, 1)


# speedup vs baseline: 1.5376x; 1.5376x over previous
import functools

import jax
import jax.numpy as jnp
from jax import lax
from jax.experimental import pallas as pl
from jax.experimental.pallas import tpu as pltpu
from jax.experimental.pallas import tpu_sc as plsc

NC = 2
NS = 16
NW = NC * NS

D = 64
DP = 128
CH = 128
NBUF = 4
L = 16


@functools.partial(jax.jit, static_argnames=("n_h", "n_b"))
def _gather(idx3, table_p, *, n_h, n_b):
    n_rows = n_h * n_b
    per_w = n_rows // NW
    n_ch = per_w // CH
    mesh = plsc.VectorSubcoreMesh(core_axis_name="c", subcore_axis_name="s")

    @functools.partial(
        pl.kernel,
        out_type=jax.ShapeDtypeStruct((n_h, D, n_b), jnp.float32),
        mesh=mesh,
        scratch_types=[
            pltpu.VMEM((n_ch, CH), jnp.int32),
            [pltpu.VMEM((CH, DP), jnp.float32) for _ in range(NBUF)],
            [pltpu.VMEM((D, CH), jnp.float32) for _ in range(NBUF)],
            [pltpu.SemaphoreType.DMA for _ in range(NBUF)],
            [pltpu.SemaphoreType.DMA for _ in range(NBUF)],
        ],
        compiler_params=pltpu.CompilerParams(needs_layout_passes=False),
    )
    def k(table_hbm, idx_hbm, out_hbm, idx_v, gbufs, tbufs, gsems, wsems):
        wid = lax.axis_index("s") * NC + lax.axis_index("c")
        w_base = wid * per_w

        pltpu.sync_copy(idx_hbm.at[wid], idx_v)

        def gather_chunk(g, b):
            return pltpu.make_async_copy(
                table_hbm.at[idx_v.at[g]], gbufs[b], gsems[b])

        def write_chunk(g, b):
            row = w_base + g * CH
            return pltpu.make_async_copy(
                tbufs[b],
                out_hbm.at[row // n_b, :, pl.ds(row % n_b, CH)],
                wsems[b])

        def transpose_chunk(b):
            gb, tb = gbufs[b], tbufs[b]
            lane = lax.iota(jnp.int32, L)

            @plsc.parallel_loop(0, (D // L) * (CH // L), step=1)
            def _(i):
                d0 = (i & ((D // L) - 1)) * L
                rows = lane + (i >> 2) * L
                for s in range(L):
                    colv = ((lane + s) & (L - 1)) + d0
                    val = plsc.load_gather(gb, [rows, colv])
                    plsc.store_scatter(tb, [colv, rows], val)

        for b in range(NBUF - 1):
            gather_chunk(b, b).start()

        def outer(i, carry):
            so = i * NBUF
            for b in range(NBUF):
                g = so + b
                @pl.when(g >= NBUF)
                def _():
                    write_chunk(g - NBUF, b).wait()

                gather_chunk(g, b).wait()
                bn = (b + NBUF - 1) % NBUF

                @pl.when(g + NBUF - 1 < n_ch)
                def _():
                    gather_chunk(g + NBUF - 1, bn).start()

                transpose_chunk(b)
                write_chunk(g, b).start()
            return carry

        lax.fori_loop(0, n_ch // NBUF, outer, 0)

        for j in range(NBUF):
            g = n_ch - NBUF + j
            write_chunk(g, g % NBUF).wait()

    return k(table_p, idx3)


def kernel(imputs, table):
    b, h = imputs.shape
    n_rows = b * h
    per_w = n_rows // NW
    idx3 = imputs.T.reshape(NW, per_w // CH, CH).astype(jnp.int32)
    table_p = jnp.pad(table, ((0, 0), (0, DP - D)))
    out = _gather(idx3, table_p, n_h=h, n_b=b)
    return out.transpose(2, 0, 1)
